# baseline (speedup 1.0000x reference)
import functools

import jax
import jax.numpy as jnp
import numpy as np
from jax import lax
from jax.experimental import pallas as pl
from jax.experimental.pallas import tpu as pltpu
from jax.experimental.pallas import tpu_sc as plsc

_B = 16384
_F = 26
_H = 16
_E = 2600000
_NC = 2
_NS = 16
_NW = _NC * _NS
_CHUNK = 128
_NCHUNKS = _B // _CHUNK
_CPW = _NCHUNKS // _NW
_G = _CHUNK // 16

_RC = 2000
_RCP = _RC + 1
_RNB = _E // _RC
_RPW = _RNB // _NW
_RXT = _RNB - _RPW * _NW


def _relayout_body(embt_hbm, out_hbm, in_v, out_v, sem):
    c = lax.axis_index("c")
    s = lax.axis_index("s")
    wid = s * _NC + c
    lane = lax.iota(jnp.int32, 16)

    def blk(b, carry):
        col0 = (wid + b * _NW) * _RC
        handles = [
            pltpu.async_copy(embt_hbm.at[h].at[pl.ds(col0, _RC)],
                             in_v.at[h].at[pl.ds(0, _RC)], sem)
            for h in range(_H)
        ]
        for hnd in handles:
            hnd.wait()

        def rows(r0, c2):
            for u in range(8):
                r = r0 * 8 + u
                out_v[r, :] = plsc.load_gather(
                    in_v, [lane, jnp.full((16,), r, jnp.int32)])
            return c2

        lax.fori_loop(0, _RC // 8, rows, 0)
        pltpu.sync_copy(out_v, out_hbm.at[pl.ds(col0, _RC)])
        return carry

    nblk = _RPW + jnp.where(wid < _RXT, 1, 0)
    lax.fori_loop(0, nblk, blk, 0)


@functools.cache
def _build_relayout():
    return pl.kernel(
        _relayout_body,
        mesh=plsc.VectorSubcoreMesh(core_axis_name="c", subcore_axis_name="s"),
        compiler_params=pltpu.CompilerParams(
            needs_layout_passes=False, use_tc_tiling_on_sc=False),
        out_type=jax.ShapeDtypeStruct((_E, _H), jnp.float32),
        scratch_types=[
            pltpu.VMEM((_H, _RCP), jnp.float32),
            pltpu.VMEM((_RC, _H), jnp.float32),
            pltpu.SemaphoreType.DMA,
        ],
    )


def _fm_body(xot_hbm, fc_hbm, emb_hbm, bias_hbm, out_hbm,
             idx0, idx1, rows0, rows1, lin0, lin1, bias_v, out_v, tbuf,
             sem0, sem1):
    c = lax.axis_index("c")
    s = lax.axis_index("s")
    wid = s * _NC + c

    pltpu.sync_copy(bias_hbm, bias_v)
    bias_vec = bias_v[...]
    lane = lax.iota(jnp.int32, 16)

    idx_bufs = (idx0, idx1)
    rows_bufs = (rows0, rows1)
    lin_bufs = (lin0, lin1)
    sems = (sem0, sem1)

    def fire(ci, k):
        chunk = wid * _CPW + ci
        pltpu.sync_copy(xot_hbm.at[:, pl.ds(chunk * _CHUNK, _CHUNK)],
                        idx_bufs[k])
        handles = []
        for f in range(_F):
            handles.append(pltpu.async_copy(
                emb_hbm.at[idx_bufs[k].at[f]], rows_bufs[k].at[f], sems[k]))
            handles.append(pltpu.async_copy(
                fc_hbm.at[idx_bufs[k].at[f]], lin_bufs[k].at[f], sems[k]))
        return handles

    def compute(ci, k):
        rows_v = rows_bufs[k]
        lin_v = lin_bufs[k]

        def group(g, carry):
            lin_acc = bias_vec
            for f in range(_F):
                lin_acc = lin_acc + lin_v[f, pl.ds(g * 16, 16)]

            def sample(l, c4):
                j = g * 16 + l
                a = jnp.zeros((16,), jnp.float32)
                q = jnp.zeros((16,), jnp.float32)
                for f in range(_F):
                    v = rows_v[f, j, :]
                    a = a + v
                    q = q + v * v
                tbuf[l, :] = a * a - q
                return c4

            lax.fori_loop(0, 16, sample, 0)
            acc = jnp.zeros((16,), jnp.float32)
            for h in range(16):
                col = plsc.load_gather(tbuf, [lane, jnp.full((16,), h, jnp.int32)])
                acc = acc + col
            z = 0.5 * acc + lin_acc
            out_v[pl.ds(g * 16, 16)] = 1.0 / (1.0 + jnp.exp(-z))
            return carry

        lax.fori_loop(0, _G, group, 0)
        chunk = wid * _CPW + ci
        pltpu.sync_copy(out_v, out_hbm.at[pl.ds(chunk * _CHUNK, _CHUNK)])

    handles = fire(0, 0)
    for ci in range(_CPW):
        nxt = fire(ci + 1, (ci + 1) % 2) if ci + 1 < _CPW else None
        for h in handles:
            h.wait()
        compute(ci, ci % 2)
        handles = nxt


@functools.cache
def _build_fm_kernel():
    return pl.kernel(
        _fm_body,
        mesh=plsc.VectorSubcoreMesh(core_axis_name="c", subcore_axis_name="s"),
        compiler_params=pltpu.CompilerParams(
            needs_layout_passes=False, use_tc_tiling_on_sc=False),
        out_type=jax.ShapeDtypeStruct((_B,), jnp.float32),
        scratch_types=[
            pltpu.VMEM((_F, _CHUNK), jnp.int32),
            pltpu.VMEM((_F, _CHUNK), jnp.int32),
            pltpu.VMEM((_F, _CHUNK, _H), jnp.float32),
            pltpu.VMEM((_F, _CHUNK, _H), jnp.float32),
            pltpu.VMEM((_F, _CHUNK), jnp.float32),
            pltpu.VMEM((_F, _CHUNK), jnp.float32),
            pltpu.VMEM((16,), jnp.float32),
            pltpu.VMEM((_CHUNK,), jnp.float32),
            pltpu.VMEM((16, 16), jnp.float32),
            pltpu.SemaphoreType.DMA,
            pltpu.SemaphoreType.DMA,
        ],
    )


def kernel(x, fc_w, embed_w, bias):
    offs = np.arange(_F, dtype=np.int32) * 100000
    xo_t = (x.astype(jnp.int32) + jnp.asarray(offs)[None, :]).T
    emb_t = embed_w.T
    fc_flat = fc_w.reshape(-1)
    bias16 = jnp.broadcast_to(bias, (16,)).astype(jnp.float32)
    emb_rm = _build_relayout()(emb_t)
    return _build_fm_kernel()(xo_t, fc_flat, emb_rm, bias16)


# speedup vs baseline: 3.2594x; 3.2594x over previous
import functools

import jax
import jax.numpy as jnp
import numpy as np
from jax import lax
from jax.experimental import pallas as pl
from jax.experimental.pallas import tpu as pltpu
from jax.experimental.pallas import tpu_sc as plsc

_B = 16384
_F = 26
_H = 16
_E = 2600000
_NC = 2
_NS = 16
_NW = _NC * _NS
_CHUNK = 128
_NCHUNKS = _B // _CHUNK
_CPW = _NCHUNKS // _NW
_G = _CHUNK // 16

_RC = 2000
_RCP = _RC + 1
_RNB = _E // _RC
_RPW = _RNB // _NW
_RXT = _RNB - _RPW * _NW


def _relayout_body(embt_hbm, out_hbm, in_v, out_v, sem):
    c = lax.axis_index("c")
    s = lax.axis_index("s")
    wid = s * _NC + c
    lane = lax.iota(jnp.int32, 16)

    def blk(b, carry):
        col0 = (wid + b * _NW) * _RC
        handles = [
            pltpu.async_copy(embt_hbm.at[h].at[pl.ds(col0, _RC)],
                             in_v.at[h].at[pl.ds(0, _RC)], sem)
            for h in range(_H)
        ]
        for hnd in handles:
            hnd.wait()

        def rows(r0, c2):
            for u in range(8):
                r = r0 * 8 + u
                out_v[r, :] = plsc.load_gather(
                    in_v, [lane, jnp.full((16,), r, jnp.int32)])
            return c2

        lax.fori_loop(0, _RC // 8, rows, 0)
        pltpu.sync_copy(out_v, out_hbm.at[pl.ds(col0, _RC)])
        return carry

    nblk = _RPW + jnp.where(wid < _RXT, 1, 0)
    lax.fori_loop(0, nblk, blk, 0)


@functools.cache
def _build_relayout():
    return pl.kernel(
        _relayout_body,
        mesh=plsc.VectorSubcoreMesh(core_axis_name="c", subcore_axis_name="s"),
        compiler_params=pltpu.CompilerParams(
            needs_layout_passes=False, use_tc_tiling_on_sc=False),
        out_type=jax.ShapeDtypeStruct((_E, _H), jnp.float32),
        scratch_types=[
            pltpu.VMEM((_H, _RCP), jnp.float32),
            pltpu.VMEM((_RC, _H), jnp.float32),
            pltpu.SemaphoreType.DMA,
        ],
    )


def _fm_body(xot_hbm, fc_hbm, emb_hbm, bias_hbm, out_hbm,
             idx0, idx1, rows0, rows1, lin0, lin1, bias_v, out_v, tbuf,
             sem0, sem1):
    c = lax.axis_index("c")
    s = lax.axis_index("s")
    wid = s * _NC + c

    pltpu.sync_copy(bias_hbm, bias_v)
    bias_vec = bias_v[...]
    lane = lax.iota(jnp.int32, 16)

    idx_bufs = (idx0, idx1)
    rows_bufs = (rows0, rows1)
    lin_bufs = (lin0, lin1)
    sems = (sem0, sem1)

    def fire(ci, k):
        chunk = wid * _CPW + ci
        pltpu.sync_copy(xot_hbm.at[:, pl.ds(chunk * _CHUNK, _CHUNK)],
                        idx_bufs[k])
        handles = []
        for f in range(_F):
            handles.append(pltpu.async_copy(
                emb_hbm.at[idx_bufs[k].at[f]], rows_bufs[k].at[f], sems[k]))
            handles.append(pltpu.async_copy(
                fc_hbm.at[idx_bufs[k].at[f]], lin_bufs[k].at[f], sems[k]))
        return handles

    def compute(ci, k):
        rows_v = rows_bufs[k]
        lin_v = lin_bufs[k]

        def group(g, carry):
            lin_acc = bias_vec
            for f in range(_F):
                lin_acc = lin_acc + lin_v[f, pl.ds(g * 16, 16)]

            def sample(l, c4):
                j = g * 16 + l
                a = jnp.zeros((16,), jnp.float32)
                q = jnp.zeros((16,), jnp.float32)
                for f in range(_F):
                    v = rows_v[f, j, :]
                    a = a + v
                    q = q + v * v
                tbuf[l, :] = a * a - q
                return c4

            lax.fori_loop(0, 16, sample, 0)
            acc = jnp.zeros((16,), jnp.float32)
            for h in range(16):
                col = plsc.load_gather(tbuf, [lane, jnp.full((16,), h, jnp.int32)])
                acc = acc + col
            z = 0.5 * acc + lin_acc
            out_v[pl.ds(g * 16, 16)] = 1.0 / (1.0 + jnp.exp(-z))
            return carry

        lax.fori_loop(0, _G, group, 0)
        chunk = wid * _CPW + ci
        pltpu.sync_copy(out_v, out_hbm.at[pl.ds(chunk * _CHUNK, _CHUNK)])

    handles = fire(0, 0)
    for ci in range(_CPW):
        nxt = fire(ci + 1, (ci + 1) % 2) if ci + 1 < _CPW else None
        for h in handles:
            h.wait()
        compute(ci, ci % 2)
        handles = nxt


@functools.cache
def _build_fm_kernel():
    return pl.kernel(
        _fm_body,
        mesh=plsc.VectorSubcoreMesh(core_axis_name="c", subcore_axis_name="s"),
        compiler_params=pltpu.CompilerParams(
            needs_layout_passes=False, use_tc_tiling_on_sc=False),
        out_type=jax.ShapeDtypeStruct((_B,), jnp.float32),
        scratch_types=[
            pltpu.VMEM((_F, _CHUNK), jnp.int32),
            pltpu.VMEM((_F, _CHUNK), jnp.int32),
            pltpu.VMEM((_F, _CHUNK, _H), jnp.float32),
            pltpu.VMEM((_F, _CHUNK, _H), jnp.float32),
            pltpu.VMEM((_F, _CHUNK), jnp.float32),
            pltpu.VMEM((_F, _CHUNK), jnp.float32),
            pltpu.VMEM((16,), jnp.float32),
            pltpu.VMEM((_CHUNK,), jnp.float32),
            pltpu.VMEM((16, 16), jnp.float32),
            pltpu.SemaphoreType.DMA,
            pltpu.SemaphoreType.DMA,
        ],
    )


def kernel(x, fc_w, embed_w, bias):
    offs = np.arange(_F, dtype=np.int32) * 100000
    xo_t = (x.astype(jnp.int32) + jnp.asarray(offs)[None, :]).T
    fc_flat = fc_w.reshape(-1)
    bias16 = jnp.broadcast_to(bias, (16,)).astype(jnp.float32)
    return _build_fm_kernel()(xo_t, fc_flat, embed_w, bias16)
